# baseline (speedup 1.0000x reference)
import functools

import jax
import jax.numpy as jnp
from jax import lax
from jax.experimental import pallas as pl
from jax.experimental.pallas import tpu as pltpu
from jax.experimental.pallas import tpu_sc as plsc

D = 64
L = 16


def _make_sc_gather(B, NV):
    info = plsc.get_sparse_core_info()
    NC, NS = info.num_cores, info.num_subcores
    NW = NC * NS
    b_per_w = B // NW
    CH = 128
    n_chunks = b_per_w // CH
    mesh = plsc.VectorSubcoreMesh(core_axis_name="c", subcore_axis_name="s")

    @functools.partial(
        pl.kernel,
        mesh=mesh,
        out_type=jax.ShapeDtypeStruct((B, D), jnp.float32),
        scratch_types=[
            pltpu.VMEM((b_per_w,), jnp.int32),
            pltpu.VMEM((2, CH, D), jnp.float32),
            pltpu.SemaphoreType.DMA,
            pltpu.SemaphoreType.DMA,
        ],
    )
    def gather_kernel(ids_hbm, tab_hbm, out_hbm, ids_v, rows_v, sem0, sem1):
        wid = lax.axis_index("s") * NC + lax.axis_index("c")
        base = wid * b_per_w
        pltpu.async_copy(ids_hbm.at[pl.ds(base, b_per_w)], ids_v, sem0).wait()
        sems = (sem0, sem1)

        def issue(c, s):
            sem = sems[s]
            off = c * CH

            def body(k, _):
                vec = ids_v[pl.ds(off + k * L, L)]
                for lane in range(L):
                    r = k * L + lane
                    pltpu.async_copy(tab_hbm.at[pl.ds(vec[lane], 1)],
                                     rows_v.at[s, pl.ds(r, 1)], sem)
                return 0

            lax.fori_loop(0, CH // L, body, 0)

        def drain_and_flush(c, s):
            pltpu.make_async_copy(tab_hbm.at[pl.ds(0, CH)], rows_v.at[s],
                                  sems[s]).wait()
            off = c * CH
            pltpu.sync_copy(rows_v.at[s], out_hbm.at[pl.ds(base + off, CH)])

        issue(0, 0)
        for c in range(1, n_chunks):
            issue(c, c % 2)
            drain_and_flush(c - 1, (c - 1) % 2)
        drain_and_flush(n_chunks - 1, (n_chunks - 1) % 2)

    return gather_kernel


def _tc_gather_body(ids_ref, tab_ref, out_ref, sem):
    g = pl.program_id(0)
    chunk = out_ref.shape[0]

    def issue(i, _):
        rid = ids_ref[g * chunk + i]
        pltpu.async_copy(tab_ref.at[pl.ds(rid, 1)], out_ref.at[pl.ds(i, 1)],
                         sem)
        return 0

    lax.fori_loop(0, chunk, issue, 0, unroll=8)
    pltpu.make_async_copy(tab_ref.at[pl.ds(0, chunk)], out_ref, sem).wait()


def _tc_gather(ids, table, chunk=1024):
    B = ids.shape[0]
    return pl.pallas_call(
        _tc_gather_body,
        grid_spec=pltpu.PrefetchScalarGridSpec(
            num_scalar_prefetch=1,
            grid=(B // chunk,),
            in_specs=[pl.BlockSpec(memory_space=pltpu.MemorySpace.HBM)],
            out_specs=pl.BlockSpec((chunk, D), lambda i, ids: (i, 0)),
            scratch_shapes=[pltpu.SemaphoreType.DMA],
        ),
        out_shape=jax.ShapeDtypeStruct((B, D), jnp.float32),
    )(ids, table)


def _mlp_body(ue, me, w1, b1, w2, b2, w3, b3, wo, bo, out):
    x1 = (jnp.dot(ue[...], w1[0:D, :], preferred_element_type=jnp.float32)
          + jnp.dot(me[...], w1[D:2 * D, :], preferred_element_type=jnp.float32))
    h1 = jnp.maximum(x1 + b1[...], 0.0)
    h2 = jnp.maximum(
        jnp.dot(h1, w2[...], preferred_element_type=jnp.float32) + b2[...], 0.0)
    h3 = jnp.maximum(
        jnp.dot(h2, w3[...], preferred_element_type=jnp.float32) + b3[...], 0.0)
    out[...] = jnp.dot(h3, wo[...], preferred_element_type=jnp.float32) + bo[...]


def _mlp(ue, me, W1, b1, W2, b2, W3, b3, Wout, bout):
    B = ue.shape[0]
    BM = min(4096, B)
    return pl.pallas_call(
        _mlp_body,
        grid=(B // BM,),
        in_specs=[
            pl.BlockSpec((BM, D), lambda i: (i, 0)),
            pl.BlockSpec((BM, D), lambda i: (i, 0)),
            pl.BlockSpec((2 * D, 128), lambda i: (0, 0)),
            pl.BlockSpec((1, 128), lambda i: (0, 0)),
            pl.BlockSpec((128, 64), lambda i: (0, 0)),
            pl.BlockSpec((1, 64), lambda i: (0, 0)),
            pl.BlockSpec((64, 32), lambda i: (0, 0)),
            pl.BlockSpec((1, 32), lambda i: (0, 0)),
            pl.BlockSpec((32, 1), lambda i: (0, 0)),
            pl.BlockSpec((1, 1), lambda i: (0, 0)),
        ],
        out_specs=pl.BlockSpec((BM, 1), lambda i: (i, 0)),
        out_shape=jax.ShapeDtypeStruct((B, 1), jnp.float32),
    )(ue, me, W1, b1.reshape(1, 128), W2, b2.reshape(1, 64),
      W3, b3.reshape(1, 32), Wout, bout.reshape(1, 1))


def kernel(user_ids, movie_ids, user_table, movie_table,
           W1, b1, W2, b2, W3, b3, Wout, bout):
    B = user_ids.shape[0]
    uid = user_ids.astype(jnp.int32)
    mid = movie_ids.astype(jnp.int32)

    sc_gather = _make_sc_gather(B, movie_table.shape[0])
    me = sc_gather(mid, movie_table)
    ue = _tc_gather(uid, user_table)
    out = _mlp(ue, me, W1, b1, W2, b2, W3, b3, Wout, bout)
    return out[:, 0]


# speedup vs baseline: 1.1607x; 1.1607x over previous
import functools

import jax
import jax.numpy as jnp
from jax import lax
from jax.experimental import pallas as pl
from jax.experimental.pallas import tpu as pltpu
from jax.experimental.pallas import tpu_sc as plsc

D = 64
L = 16


def _make_gather(B, NU, NM):
    info = plsc.get_sparse_core_info()
    NC, NS = info.num_cores, info.num_subcores
    NW = NC * NS
    b_per_w = B // NW
    CH = 128
    n_chunks = b_per_w // CH
    mesh = plsc.VectorSubcoreMesh(core_axis_name="c", subcore_axis_name="s")

    @functools.partial(
        pl.kernel,
        mesh=mesh,
        out_type=[
            jax.ShapeDtypeStruct((B, D), jnp.float32),
            jax.ShapeDtypeStruct((B, D), jnp.float32),
        ],
        scratch_types=[
            pltpu.VMEM((b_per_w,), jnp.int32),
            pltpu.VMEM((b_per_w,), jnp.int32),
            pltpu.VMEM((2, CH, D), jnp.float32),
            pltpu.VMEM((2, CH, D), jnp.float32),
            pltpu.SemaphoreType.DMA,
            pltpu.SemaphoreType.DMA,
        ],
    )
    def gather_kernel(uid_hbm, mid_hbm, utab_hbm, mtab_hbm, uout_hbm,
                      mout_hbm, uids_v, mids_v, urows_v, mrows_v, sem0, sem1):
        wid = lax.axis_index("s") * NC + lax.axis_index("c")
        base = wid * b_per_w
        pltpu.async_copy(uid_hbm.at[pl.ds(base, b_per_w)], uids_v, sem0).wait()
        pltpu.async_copy(mid_hbm.at[pl.ds(base, b_per_w)], mids_v, sem1).wait()
        sems = (sem0, sem1)

        def issue(c, s):
            sem = sems[s]
            off = c * CH

            def body(k, _):
                uvec = uids_v[pl.ds(off + k * L, L)]
                mvec = mids_v[pl.ds(off + k * L, L)]
                for lane in range(L):
                    r = k * L + lane
                    pltpu.async_copy(utab_hbm.at[pl.ds(uvec[lane], 1)],
                                     urows_v.at[s, pl.ds(r, 1)], sem)
                    pltpu.async_copy(mtab_hbm.at[pl.ds(mvec[lane], 1)],
                                     mrows_v.at[s, pl.ds(r, 1)], sem)
                return 0

            lax.fori_loop(0, CH // L, body, 0, unroll=2)

        def drain_and_flush(c, s):
            sem = sems[s]
            pltpu.make_async_copy(utab_hbm.at[pl.ds(0, CH)], urows_v.at[s],
                                  sem).wait()
            pltpu.make_async_copy(mtab_hbm.at[pl.ds(0, CH)], mrows_v.at[s],
                                  sem).wait()
            off = c * CH
            pltpu.sync_copy(urows_v.at[s], uout_hbm.at[pl.ds(base + off, CH)])
            pltpu.sync_copy(mrows_v.at[s], mout_hbm.at[pl.ds(base + off, CH)])

        issue(0, 0)
        for c in range(1, n_chunks):
            issue(c, c % 2)
            drain_and_flush(c - 1, (c - 1) % 2)
        drain_and_flush(n_chunks - 1, (n_chunks - 1) % 2)

    return gather_kernel


def _mlp_body(ue, me, w1, b1, w2, b2, w3, b3, wo, bo, out):
    x1 = (jnp.dot(ue[...], w1[0:D, :], preferred_element_type=jnp.float32)
          + jnp.dot(me[...], w1[D:2 * D, :], preferred_element_type=jnp.float32))
    h1 = jnp.maximum(x1 + b1[...], 0.0)
    h2 = jnp.maximum(
        jnp.dot(h1, w2[...], preferred_element_type=jnp.float32) + b2[...], 0.0)
    h3 = jnp.maximum(
        jnp.dot(h2, w3[...], preferred_element_type=jnp.float32) + b3[...], 0.0)
    out[...] = jnp.dot(h3, wo[...], preferred_element_type=jnp.float32) + bo[...]


def _mlp(ue, me, W1, b1, W2, b2, W3, b3, Wout, bout):
    B = ue.shape[0]
    BM = min(8192, B)
    return pl.pallas_call(
        _mlp_body,
        grid=(B // BM,),
        in_specs=[
            pl.BlockSpec((BM, D), lambda i: (i, 0)),
            pl.BlockSpec((BM, D), lambda i: (i, 0)),
            pl.BlockSpec((2 * D, 128), lambda i: (0, 0)),
            pl.BlockSpec((1, 128), lambda i: (0, 0)),
            pl.BlockSpec((128, 64), lambda i: (0, 0)),
            pl.BlockSpec((1, 64), lambda i: (0, 0)),
            pl.BlockSpec((64, 32), lambda i: (0, 0)),
            pl.BlockSpec((1, 32), lambda i: (0, 0)),
            pl.BlockSpec((32, 1), lambda i: (0, 0)),
            pl.BlockSpec((1, 1), lambda i: (0, 0)),
        ],
        out_specs=pl.BlockSpec((BM, 1), lambda i: (i, 0)),
        out_shape=jax.ShapeDtypeStruct((B, 1), jnp.float32),
    )(ue, me, W1, b1.reshape(1, 128), W2, b2.reshape(1, 64),
      W3, b3.reshape(1, 32), Wout, bout.reshape(1, 1))


def kernel(user_ids, movie_ids, user_table, movie_table,
           W1, b1, W2, b2, W3, b3, Wout, bout):
    B = user_ids.shape[0]
    NU, NM = user_table.shape[0], movie_table.shape[0]
    NSPLIT = 1
    BS = B // NSPLIT
    gather_kernel = _make_gather(BS, NU, NM)

    uid = user_ids.astype(jnp.int32)
    mid = movie_ids.astype(jnp.int32)
    halves = []
    for h in range(NSPLIT):
        sl = slice(h * BS, (h + 1) * BS)
        halves.append(gather_kernel(uid[sl], mid[sl], user_table, movie_table))
    outs = [_mlp(ue, me, W1, b1, W2, b2, W3, b3, Wout, bout)
            for ue, me in halves]
    return jnp.concatenate(outs, axis=0)[:, 0]


# speedup vs baseline: 1.1640x; 1.0028x over previous
import functools

import jax
import jax.numpy as jnp
from jax import lax
from jax.experimental import pallas as pl
from jax.experimental.pallas import tpu as pltpu
from jax.experimental.pallas import tpu_sc as plsc

D = 64
L = 16


def _make_gather(B, NU, NM):
    info = plsc.get_sparse_core_info()
    NC, NS = info.num_cores, info.num_subcores
    NW = NC * NS
    b_per_w = B // NW
    CH = 64
    n_chunks = b_per_w // CH
    mesh = plsc.VectorSubcoreMesh(core_axis_name="c", subcore_axis_name="s")

    @functools.partial(
        pl.kernel,
        mesh=mesh,
        out_type=[
            jax.ShapeDtypeStruct((B, D), jnp.float32),
            jax.ShapeDtypeStruct((B, D), jnp.float32),
        ],
        scratch_types=[
            pltpu.VMEM((b_per_w,), jnp.int32),
            pltpu.VMEM((b_per_w,), jnp.int32),
            pltpu.VMEM((2, CH, D), jnp.float32),
            pltpu.VMEM((2, CH, D), jnp.float32),
            pltpu.SemaphoreType.DMA,
            pltpu.SemaphoreType.DMA,
        ],
    )
    def gather_kernel(uid_hbm, mid_hbm, utab_hbm, mtab_hbm, uout_hbm,
                      mout_hbm, uids_v, mids_v, urows_v, mrows_v, sem0, sem1):
        wid = lax.axis_index("s") * NC + lax.axis_index("c")
        base = wid * b_per_w
        pltpu.async_copy(uid_hbm.at[pl.ds(base, b_per_w)], uids_v, sem0).wait()
        pltpu.async_copy(mid_hbm.at[pl.ds(base, b_per_w)], mids_v, sem1).wait()
        sems = (sem0, sem1)

        def issue(c, s):
            sem = sems[s]
            off = c * CH

            def body(k, _):
                uvec = uids_v[pl.ds(off + k * L, L)]
                mvec = mids_v[pl.ds(off + k * L, L)]
                for lane in range(L):
                    r = k * L + lane
                    pltpu.async_copy(utab_hbm.at[pl.ds(uvec[lane], 1)],
                                     urows_v.at[s, pl.ds(r, 1)], sem)
                    pltpu.async_copy(mtab_hbm.at[pl.ds(mvec[lane], 1)],
                                     mrows_v.at[s, pl.ds(r, 1)], sem)
                return 0

            lax.fori_loop(0, CH // L, body, 0, unroll=2)

        def drain_and_flush(c, s):
            sem = sems[s]
            pltpu.make_async_copy(utab_hbm.at[pl.ds(0, CH)], urows_v.at[s],
                                  sem).wait()
            pltpu.make_async_copy(mtab_hbm.at[pl.ds(0, CH)], mrows_v.at[s],
                                  sem).wait()
            off = c * CH
            pltpu.sync_copy(urows_v.at[s], uout_hbm.at[pl.ds(base + off, CH)])
            pltpu.sync_copy(mrows_v.at[s], mout_hbm.at[pl.ds(base + off, CH)])

        issue(0, 0)
        for c in range(1, n_chunks):
            issue(c, c % 2)
            drain_and_flush(c - 1, (c - 1) % 2)
        drain_and_flush(n_chunks - 1, (n_chunks - 1) % 2)

    return gather_kernel


def _mlp_body(ue, me, w1, b1, w2, b2, w3, b3, wo, bo, out):
    x1 = (jnp.dot(ue[...], w1[0:D, :], preferred_element_type=jnp.float32)
          + jnp.dot(me[...], w1[D:2 * D, :], preferred_element_type=jnp.float32))
    h1 = jnp.maximum(x1 + b1[...], 0.0)
    h2 = jnp.maximum(
        jnp.dot(h1, w2[...], preferred_element_type=jnp.float32) + b2[...], 0.0)
    h3 = jnp.maximum(
        jnp.dot(h2, w3[...], preferred_element_type=jnp.float32) + b3[...], 0.0)
    out[...] = jnp.dot(h3, wo[...], preferred_element_type=jnp.float32) + bo[...]


def _mlp(ue, me, W1, b1, W2, b2, W3, b3, Wout, bout):
    B = ue.shape[0]
    BM = min(8192, B)
    return pl.pallas_call(
        _mlp_body,
        grid=(B // BM,),
        in_specs=[
            pl.BlockSpec((BM, D), lambda i: (i, 0)),
            pl.BlockSpec((BM, D), lambda i: (i, 0)),
            pl.BlockSpec((2 * D, 128), lambda i: (0, 0)),
            pl.BlockSpec((1, 128), lambda i: (0, 0)),
            pl.BlockSpec((128, 64), lambda i: (0, 0)),
            pl.BlockSpec((1, 64), lambda i: (0, 0)),
            pl.BlockSpec((64, 32), lambda i: (0, 0)),
            pl.BlockSpec((1, 32), lambda i: (0, 0)),
            pl.BlockSpec((32, 1), lambda i: (0, 0)),
            pl.BlockSpec((1, 1), lambda i: (0, 0)),
        ],
        out_specs=pl.BlockSpec((BM, 1), lambda i: (i, 0)),
        out_shape=jax.ShapeDtypeStruct((B, 1), jnp.float32),
    )(ue, me, W1, b1.reshape(1, 128), W2, b2.reshape(1, 64),
      W3, b3.reshape(1, 32), Wout, bout.reshape(1, 1))


def kernel(user_ids, movie_ids, user_table, movie_table,
           W1, b1, W2, b2, W3, b3, Wout, bout):
    B = user_ids.shape[0]
    NU, NM = user_table.shape[0], movie_table.shape[0]
    NSPLIT = 1
    BS = B // NSPLIT
    gather_kernel = _make_gather(BS, NU, NM)

    uid = user_ids.astype(jnp.int32)
    mid = movie_ids.astype(jnp.int32)
    halves = []
    for h in range(NSPLIT):
        sl = slice(h * BS, (h + 1) * BS)
        halves.append(gather_kernel(uid[sl], mid[sl], user_table, movie_table))
    outs = [_mlp(ue, me, W1, b1, W2, b2, W3, b3, Wout, bout)
            for ue, me in halves]
    return jnp.concatenate(outs, axis=0)[:, 0]


# speedup vs baseline: 1.1681x; 1.0035x over previous
import functools

import jax
import jax.numpy as jnp
from jax import lax
from jax.experimental import pallas as pl
from jax.experimental.pallas import tpu as pltpu
from jax.experimental.pallas import tpu_sc as plsc

D = 64
L = 16


def _make_gather(B, NU, NM):
    info = plsc.get_sparse_core_info()
    NC, NS = info.num_cores, info.num_subcores
    NW = NC * NS
    b_per_w = B // NW
    CH = 128
    n_chunks = b_per_w // CH
    mesh = plsc.VectorSubcoreMesh(core_axis_name="c", subcore_axis_name="s")

    @functools.partial(
        pl.kernel,
        mesh=mesh,
        out_type=[
            jax.ShapeDtypeStruct((B, D), jnp.float32),
            jax.ShapeDtypeStruct((B, D), jnp.float32),
        ],
        scratch_types=[
            pltpu.VMEM((b_per_w,), jnp.int32),
            pltpu.VMEM((b_per_w,), jnp.int32),
            pltpu.VMEM((2, CH, D), jnp.float32),
            pltpu.VMEM((2, CH, D), jnp.float32),
            pltpu.SemaphoreType.DMA,
            pltpu.SemaphoreType.DMA,
        ],
    )
    def gather_kernel(uid_hbm, mid_hbm, utab_hbm, mtab_hbm, uout_hbm,
                      mout_hbm, uids_v, mids_v, urows_v, mrows_v, sem0, sem1):
        wid = lax.axis_index("s") * NC + lax.axis_index("c")
        base = wid * b_per_w
        cu = pltpu.async_copy(uid_hbm.at[pl.ds(base, b_per_w)], uids_v, sem0)
        cm = pltpu.async_copy(mid_hbm.at[pl.ds(base, b_per_w)], mids_v, sem1)
        cu.wait()
        cm.wait()
        sems = (sem0, sem1)

        def issue(c, s):
            sem = sems[s]
            off = c * CH

            def body(k, _):
                uvec = uids_v[pl.ds(off + k * L, L)]
                mvec = mids_v[pl.ds(off + k * L, L)]
                for lane in range(L):
                    r = k * L + lane
                    pltpu.async_copy(utab_hbm.at[pl.ds(uvec[lane], 1)],
                                     urows_v.at[s, pl.ds(r, 1)], sem)
                    pltpu.async_copy(mtab_hbm.at[pl.ds(mvec[lane], 1)],
                                     mrows_v.at[s, pl.ds(r, 1)], sem)
                return 0

            lax.fori_loop(0, CH // L, body, 0, unroll=2)

        def drain_and_flush(c, s):
            sem = sems[s]
            pltpu.make_async_copy(utab_hbm.at[pl.ds(0, CH)], urows_v.at[s],
                                  sem).wait()
            pltpu.make_async_copy(mtab_hbm.at[pl.ds(0, CH)], mrows_v.at[s],
                                  sem).wait()
            off = c * CH
            pltpu.sync_copy(urows_v.at[s], uout_hbm.at[pl.ds(base + off, CH)])
            pltpu.sync_copy(mrows_v.at[s], mout_hbm.at[pl.ds(base + off, CH)])

        issue(0, 0)
        for c in range(1, n_chunks):
            issue(c, c % 2)
            drain_and_flush(c - 1, (c - 1) % 2)
        drain_and_flush(n_chunks - 1, (n_chunks - 1) % 2)

    return gather_kernel


def _mlp_body(ue, me, w1, b1, w2, b2, w3, b3, wo, bo, out):
    x1 = (jnp.dot(ue[...], w1[0:D, :], preferred_element_type=jnp.float32)
          + jnp.dot(me[...], w1[D:2 * D, :], preferred_element_type=jnp.float32))
    h1 = jnp.maximum(x1 + b1[...], 0.0)
    h2 = jnp.maximum(
        jnp.dot(h1, w2[...], preferred_element_type=jnp.float32) + b2[...], 0.0)
    h3 = jnp.maximum(
        jnp.dot(h2, w3[...], preferred_element_type=jnp.float32) + b3[...], 0.0)
    out[...] = jnp.dot(h3, wo[...], preferred_element_type=jnp.float32) + bo[...]


def _mlp(ue, me, W1, b1, W2, b2, W3, b3, Wout, bout):
    B = ue.shape[0]
    BM = min(8192, B)
    return pl.pallas_call(
        _mlp_body,
        grid=(B // BM,),
        in_specs=[
            pl.BlockSpec((BM, D), lambda i: (i, 0)),
            pl.BlockSpec((BM, D), lambda i: (i, 0)),
            pl.BlockSpec((2 * D, 128), lambda i: (0, 0)),
            pl.BlockSpec((1, 128), lambda i: (0, 0)),
            pl.BlockSpec((128, 64), lambda i: (0, 0)),
            pl.BlockSpec((1, 64), lambda i: (0, 0)),
            pl.BlockSpec((64, 32), lambda i: (0, 0)),
            pl.BlockSpec((1, 32), lambda i: (0, 0)),
            pl.BlockSpec((32, 1), lambda i: (0, 0)),
            pl.BlockSpec((1, 1), lambda i: (0, 0)),
        ],
        out_specs=pl.BlockSpec((BM, 1), lambda i: (i, 0)),
        out_shape=jax.ShapeDtypeStruct((B, 1), jnp.float32),
    )(ue, me, W1, b1.reshape(1, 128), W2, b2.reshape(1, 64),
      W3, b3.reshape(1, 32), Wout, bout.reshape(1, 1))


def kernel(user_ids, movie_ids, user_table, movie_table,
           W1, b1, W2, b2, W3, b3, Wout, bout):
    B = user_ids.shape[0]
    NU, NM = user_table.shape[0], movie_table.shape[0]
    gather_kernel = _make_gather(B, NU, NM)

    uid = user_ids.astype(jnp.int32)
    mid = movie_ids.astype(jnp.int32)
    ue, me = gather_kernel(uid, mid, user_table, movie_table)
    out = _mlp(ue, me, W1, b1, W2, b2, W3, b3, Wout, bout)
    return out[:, 0]
